# baseline (speedup 1.0000x reference)
import functools

import jax
import jax.numpy as jnp
from jax import lax
from jax.experimental import pallas as pl
from jax.experimental.pallas import tpu as pltpu
from jax.experimental.pallas import tpu_sc as plsc

B = 16384
D = 64
CHUNK = 128


def _make_sc_gather():
  info = plsc.get_sparse_core_info()
  nc, ns = info.num_cores, info.num_subcores
  nw = nc * ns
  b_per_w = B // nw
  n_chunks = b_per_w // CHUNK

  mesh = plsc.VectorSubcoreMesh(core_axis_name="c", subcore_axis_name="s")

  @functools.partial(
      pl.kernel,
      mesh=mesh,
      compiler_params=pltpu.CompilerParams(use_tc_tiling_on_sc=False),
      out_type=[
          jax.ShapeDtypeStruct((B, 2 * D), jnp.float32),
          jax.ShapeDtypeStruct((B,), jnp.float32),
          jax.ShapeDtypeStruct((B,), jnp.float32),
      ],
      scratch_types=[
          pltpu.VMEM((n_chunks, CHUNK), jnp.int32),
          pltpu.VMEM((n_chunks, CHUNK), jnp.int32),
          pltpu.VMEM((b_per_w, D), jnp.float32),
          pltpu.VMEM((b_per_w, D), jnp.float32),
          pltpu.VMEM((b_per_w,), jnp.float32),
          pltpu.VMEM((b_per_w,), jnp.float32),
          pltpu.SemaphoreType.DMA,
      ],
  )
  def gather_kernel(uidx_hbm, iidx_hbm, emb_hbm, bias_hbm,
                    out_x, out_ub, out_ib,
                    uidx_v, iidx_v, ue_v, ie_v, ub_v, ib_v, sem):
    wid = lax.axis_index("s") * nc + lax.axis_index("c")
    base = wid * b_per_w
    row0 = wid * n_chunks

    pltpu.sync_copy(uidx_hbm.at[pl.ds(row0, n_chunks)], uidx_v)
    pltpu.sync_copy(iidx_hbm.at[pl.ds(row0, n_chunks)], iidx_v)

    copies = []
    for j in range(n_chunks):
      sl = pl.ds(j * CHUNK, CHUNK)
      copies.append(pltpu.async_copy(emb_hbm.at[uidx_v.at[j]], ue_v.at[sl], sem))
      copies.append(pltpu.async_copy(emb_hbm.at[iidx_v.at[j]], ie_v.at[sl], sem))
      copies.append(pltpu.async_copy(bias_hbm.at[uidx_v.at[j]], ub_v.at[sl], sem))
      copies.append(pltpu.async_copy(bias_hbm.at[iidx_v.at[j]], ib_v.at[sl], sem))
    for c in copies:
      c.wait()

    out_sl = pl.ds(base, b_per_w)
    pltpu.sync_copy(ue_v, out_x.at[out_sl, pl.ds(0, D)])
    pltpu.sync_copy(ie_v, out_x.at[out_sl, pl.ds(D, D)])
    pltpu.sync_copy(ub_v, out_ub.at[out_sl])
    pltpu.sync_copy(ib_v, out_ib.at[out_sl])

  return gather_kernel


_sc_gather = _make_sc_gather()


def _mlp_body(xin, ub, ib, w0, b0, w1, b1, w2, b2, w3, b3, w4, b4,
              wo, bo, out):
  f32 = jnp.float32
  x = jnp.dot(xin[...], w0[...], preferred_element_type=f32)
  x = jnp.maximum(x + b0[...], 0.0)
  x = jnp.maximum(jnp.dot(x, w1[...], preferred_element_type=f32) + b1[...], 0.0)
  x = jnp.maximum(jnp.dot(x, w2[...], preferred_element_type=f32) + b2[...], 0.0)
  x = jnp.maximum(jnp.dot(x, w3[...], preferred_element_type=f32) + b3[...], 0.0)
  x = jnp.maximum(jnp.dot(x, w4[...], preferred_element_type=f32) + b4[...], 0.0)
  o = jnp.dot(x, wo[...], preferred_element_type=f32)
  out[...] = o + bo[...] + ub[...] + ib[...]


def _mlp(x, ub, ib, w0, b0, w1, b1, w2, b2, w3, b3, w4, b4, wo, bo,
         blk=2048):
  grid = (B // blk,)

  def data_spec(n):
    return pl.BlockSpec((blk, n), lambda i: (i, 0))

  def w_spec(m, n):
    return pl.BlockSpec((m, n), lambda i: (0, 0))

  return pl.pallas_call(
      _mlp_body,
      grid=grid,
      in_specs=[
          data_spec(2 * D), data_spec(1), data_spec(1),
          w_spec(2 * D, 128), w_spec(1, 128),
          w_spec(128, 256), w_spec(1, 256),
          w_spec(256, 128), w_spec(1, 128),
          w_spec(128, 64), w_spec(1, 64),
          w_spec(64, 32), w_spec(1, 32),
          w_spec(32, 1), w_spec(1, 1),
      ],
      out_specs=data_spec(1),
      out_shape=jax.ShapeDtypeStruct((B, 1), jnp.float32),
      compiler_params=pltpu.CompilerParams(
          dimension_semantics=("arbitrary",),
      ),
  )(x, ub, ib, w0, b0, w1, b1, w2, b2, w3, b3, w4, b4, wo, bo)


def kernel(user_idx, item_idx, user_embed, item_embed, user_bias, item_bias,
           W0, b0, W1, b1, W2, b2, W3, b3, W4, b4, Wo, bo):
  uidx = user_idx.astype(jnp.int32).reshape(B // CHUNK, CHUNK)
  iidx = (item_idx.astype(jnp.int32) + 100000).reshape(B // CHUNK, CHUNK)

  emb = jnp.concatenate([user_embed, item_embed], axis=0)
  bias = jnp.concatenate([user_bias.reshape(-1), item_bias.reshape(-1)])
  x, ub, ib = _sc_gather(uidx, iidx, emb, bias)
  ub = ub.reshape(B, 1)
  ib = ib.reshape(B, 1)

  out = _mlp(x, ub, ib, W0, b0.reshape(1, -1),
             W1, b1.reshape(1, -1), W2, b2.reshape(1, -1),
             W3, b3.reshape(1, -1), W4, b4.reshape(1, -1),
             Wo, bo.reshape(1, 1))
  return out


# speedup vs baseline: 1.3745x; 1.3745x over previous
import functools

import jax
import jax.numpy as jnp
from jax import lax
from jax.experimental import pallas as pl
from jax.experimental.pallas import tpu as pltpu
from jax.experimental.pallas import tpu_sc as plsc

B = 16384
D = 64
CHUNK = 128


def _make_sc_gather():
  info = plsc.get_sparse_core_info()
  nc, ns = info.num_cores, info.num_subcores
  nw = nc * ns
  b_per_w = B // nw
  n_chunks = b_per_w // CHUNK

  mesh = plsc.VectorSubcoreMesh(core_axis_name="c", subcore_axis_name="s")

  @functools.partial(
      pl.kernel,
      mesh=mesh,
      compiler_params=pltpu.CompilerParams(use_tc_tiling_on_sc=False),
      out_type=[
          jax.ShapeDtypeStruct((B, 2 * D), jnp.float32),
          jax.ShapeDtypeStruct((B,), jnp.float32),
          jax.ShapeDtypeStruct((B,), jnp.float32),
      ],
      scratch_types=[
          pltpu.VMEM((n_chunks, CHUNK), jnp.int32),
          pltpu.VMEM((n_chunks, CHUNK), jnp.int32),
          pltpu.VMEM((b_per_w, D), jnp.float32),
          pltpu.VMEM((b_per_w, D), jnp.float32),
          pltpu.VMEM((b_per_w,), jnp.float32),
          pltpu.VMEM((b_per_w,), jnp.float32),
          pltpu.SemaphoreType.DMA,
      ],
  )
  def gather_kernel(uidx_hbm, iidx_hbm, uemb_hbm, iemb_hbm, ubias_hbm,
                    ibias_hbm, out_x, out_ub, out_ib,
                    uidx_v, iidx_v, ue_v, ie_v, ub_v, ib_v, sem):
    wid = lax.axis_index("s") * nc + lax.axis_index("c")
    base = wid * b_per_w
    row0 = wid * n_chunks

    pltpu.sync_copy(uidx_hbm.at[pl.ds(row0, n_chunks)], uidx_v)
    pltpu.sync_copy(iidx_hbm.at[pl.ds(row0, n_chunks)], iidx_v)

    copies = []
    for j in range(n_chunks):
      sl = pl.ds(j * CHUNK, CHUNK)
      copies.append(pltpu.async_copy(uemb_hbm.at[uidx_v.at[j]], ue_v.at[sl], sem))
      copies.append(pltpu.async_copy(iemb_hbm.at[iidx_v.at[j]], ie_v.at[sl], sem))
      copies.append(pltpu.async_copy(ubias_hbm.at[uidx_v.at[j]], ub_v.at[sl], sem))
      copies.append(pltpu.async_copy(ibias_hbm.at[iidx_v.at[j]], ib_v.at[sl], sem))
    for c in copies:
      c.wait()

    out_sl = pl.ds(base, b_per_w)
    pltpu.sync_copy(ue_v, out_x.at[out_sl, pl.ds(0, D)])
    pltpu.sync_copy(ie_v, out_x.at[out_sl, pl.ds(D, D)])
    pltpu.sync_copy(ub_v, out_ub.at[out_sl])
    pltpu.sync_copy(ib_v, out_ib.at[out_sl])

  return gather_kernel


_sc_gather = _make_sc_gather()


def _mlp_body(xin, ub, ib, w0, b0, w1, b1, w2, b2, w3, b3, w4, b4,
              wo, bo, out):
  f32 = jnp.float32
  x = jnp.dot(xin[...], w0[...], preferred_element_type=f32)
  x = jnp.maximum(x + b0[...], 0.0)
  x = jnp.maximum(jnp.dot(x, w1[...], preferred_element_type=f32) + b1[...], 0.0)
  x = jnp.maximum(jnp.dot(x, w2[...], preferred_element_type=f32) + b2[...], 0.0)
  x = jnp.maximum(jnp.dot(x, w3[...], preferred_element_type=f32) + b3[...], 0.0)
  x = jnp.maximum(jnp.dot(x, w4[...], preferred_element_type=f32) + b4[...], 0.0)
  o = jnp.dot(x, wo[...], preferred_element_type=f32)
  out[...] = o + bo[...] + ub[...] + ib[...]


def _mlp(x, ub, ib, w0, b0, w1, b1, w2, b2, w3, b3, w4, b4, wo, bo,
         blk=1024):
  grid = (B // blk,)

  def data_spec(n):
    return pl.BlockSpec((blk, n), lambda i: (i, 0))

  def w_spec(m, n):
    return pl.BlockSpec((m, n), lambda i: (0, 0))

  return pl.pallas_call(
      _mlp_body,
      grid=grid,
      in_specs=[
          data_spec(2 * D), data_spec(1), data_spec(1),
          w_spec(2 * D, 128), w_spec(1, 128),
          w_spec(128, 256), w_spec(1, 256),
          w_spec(256, 128), w_spec(1, 128),
          w_spec(128, 64), w_spec(1, 64),
          w_spec(64, 32), w_spec(1, 32),
          w_spec(32, 1), w_spec(1, 1),
      ],
      out_specs=data_spec(1),
      out_shape=jax.ShapeDtypeStruct((B, 1), jnp.float32),
      compiler_params=pltpu.CompilerParams(
          dimension_semantics=("arbitrary",),
      ),
  )(x, ub, ib, w0, b0, w1, b1, w2, b2, w3, b3, w4, b4, wo, bo)


def kernel(user_idx, item_idx, user_embed, item_embed, user_bias, item_bias,
           W0, b0, W1, b1, W2, b2, W3, b3, W4, b4, Wo, bo):
  uidx = user_idx.astype(jnp.int32).reshape(B // CHUNK, CHUNK)
  iidx = item_idx.astype(jnp.int32).reshape(B // CHUNK, CHUNK)

  x, ub, ib = _sc_gather(uidx, iidx, user_embed, item_embed,
                         user_bias.reshape(-1), item_bias.reshape(-1))
  ub = ub.reshape(B, 1)
  ib = ib.reshape(B, 1)

  out = _mlp(x, ub, ib, W0, b0.reshape(1, -1),
             W1, b1.reshape(1, -1), W2, b2.reshape(1, -1),
             W3, b3.reshape(1, -1), W4, b4.reshape(1, -1),
             Wo, bo.reshape(1, 1))
  return out


# speedup vs baseline: 1.4399x; 1.0475x over previous
import functools

import jax
import jax.numpy as jnp
from jax import lax
from jax.experimental import pallas as pl
from jax.experimental.pallas import tpu as pltpu
from jax.experimental.pallas import tpu_sc as plsc

B = 16384
D = 64
CHUNK = 128


def _make_sc_gather():
  info = plsc.get_sparse_core_info()
  nc, ns = info.num_cores, info.num_subcores
  nw = nc * ns
  b_per_w = B // nw
  n_chunks = b_per_w // CHUNK

  mesh = plsc.VectorSubcoreMesh(core_axis_name="c", subcore_axis_name="s")

  @functools.partial(
      pl.kernel,
      mesh=mesh,
      compiler_params=pltpu.CompilerParams(use_tc_tiling_on_sc=False),
      out_type=[
          jax.ShapeDtypeStruct((B, 2 * D), jnp.float32),
          jax.ShapeDtypeStruct((B,), jnp.float32),
          jax.ShapeDtypeStruct((B,), jnp.float32),
      ],
      scratch_types=[
          pltpu.VMEM((n_chunks, CHUNK), jnp.int32),
          pltpu.VMEM((n_chunks, CHUNK), jnp.int32),
          pltpu.VMEM((b_per_w, D), jnp.float32),
          pltpu.VMEM((b_per_w, D), jnp.float32),
          pltpu.VMEM((b_per_w,), jnp.float32),
          pltpu.VMEM((b_per_w,), jnp.float32),
          pltpu.SemaphoreType.DMA,
      ],
  )
  def gather_kernel(uidx_hbm, iidx_hbm, uemb_hbm, iemb_hbm, ubias_hbm,
                    ibias_hbm, out_x, out_ub, out_ib,
                    uidx_v, iidx_v, ue_v, ie_v, ub_v, ib_v, sem):
    wid = lax.axis_index("s") * nc + lax.axis_index("c")
    base = wid * b_per_w
    row0 = wid * n_chunks

    pltpu.sync_copy(uidx_hbm.at[pl.ds(row0, n_chunks)], uidx_v)
    pltpu.sync_copy(iidx_hbm.at[pl.ds(row0, n_chunks)], iidx_v)

    copies = []
    for j in range(n_chunks):
      sl = pl.ds(j * CHUNK, CHUNK)
      copies.append(pltpu.async_copy(uemb_hbm.at[uidx_v.at[j]], ue_v.at[sl], sem))
      copies.append(pltpu.async_copy(iemb_hbm.at[iidx_v.at[j]], ie_v.at[sl], sem))
      copies.append(pltpu.async_copy(ubias_hbm.at[uidx_v.at[j]], ub_v.at[sl], sem))
      copies.append(pltpu.async_copy(ibias_hbm.at[iidx_v.at[j]], ib_v.at[sl], sem))
    for c in copies:
      c.wait()

    out_sl = pl.ds(base, b_per_w)
    pltpu.sync_copy(ue_v, out_x.at[out_sl, pl.ds(0, D)])
    pltpu.sync_copy(ie_v, out_x.at[out_sl, pl.ds(D, D)])
    pltpu.sync_copy(ub_v, out_ub.at[out_sl])
    pltpu.sync_copy(ib_v, out_ib.at[out_sl])

  return gather_kernel


_sc_gather = _make_sc_gather()


def _mlp_body(xin, ub, ib, w0, b0, w1, b1, w2, b2, w3, b3, w4, b4,
              wo, bo, out):
  f32 = jnp.float32
  x = jnp.dot(xin[...], w0[...], preferred_element_type=f32)
  x = jnp.maximum(x + b0[...], 0.0)
  x = jnp.maximum(jnp.dot(x, w1[...], preferred_element_type=f32) + b1[...], 0.0)
  x = jnp.maximum(jnp.dot(x, w2[...], preferred_element_type=f32) + b2[...], 0.0)
  x = jnp.maximum(jnp.dot(x, w3[...], preferred_element_type=f32) + b3[...], 0.0)
  x = jnp.maximum(jnp.dot(x, w4[...], preferred_element_type=f32) + b4[...], 0.0)
  o = jnp.dot(x, wo[...], preferred_element_type=f32)
  out[...] = o + bo[...] + ub[...] + ib[...]


def _mlp(x, ub, ib, w0, b0, w1, b1, w2, b2, w3, b3, w4, b4, wo, bo,
         blk=4096):
  grid = (B // blk,)

  def data_spec(n):
    return pl.BlockSpec((blk, n), lambda i: (i, 0))

  def w_spec(m, n):
    return pl.BlockSpec((m, n), lambda i: (0, 0))

  return pl.pallas_call(
      _mlp_body,
      grid=grid,
      in_specs=[
          data_spec(2 * D), data_spec(1), data_spec(1),
          w_spec(2 * D, 128), w_spec(1, 128),
          w_spec(128, 256), w_spec(1, 256),
          w_spec(256, 128), w_spec(1, 128),
          w_spec(128, 64), w_spec(1, 64),
          w_spec(64, 32), w_spec(1, 32),
          w_spec(32, 1), w_spec(1, 1),
      ],
      out_specs=data_spec(1),
      out_shape=jax.ShapeDtypeStruct((B, 1), jnp.float32),
      compiler_params=pltpu.CompilerParams(
          dimension_semantics=("arbitrary",),
      ),
  )(x, ub, ib, w0, b0, w1, b1, w2, b2, w3, b3, w4, b4, wo, bo)


def kernel(user_idx, item_idx, user_embed, item_embed, user_bias, item_bias,
           W0, b0, W1, b1, W2, b2, W3, b3, W4, b4, Wo, bo):
  uidx = user_idx.astype(jnp.int32).reshape(B // CHUNK, CHUNK)
  iidx = item_idx.astype(jnp.int32).reshape(B // CHUNK, CHUNK)

  x, ub, ib = _sc_gather(uidx, iidx, user_embed, item_embed,
                         user_bias.reshape(-1), item_bias.reshape(-1))
  ub = ub.reshape(B, 1)
  ib = ib.reshape(B, 1)

  out = _mlp(x, ub, ib, W0, b0.reshape(1, -1),
             W1, b1.reshape(1, -1), W2, b2.reshape(1, -1),
             W3, b3.reshape(1, -1), W4, b4.reshape(1, -1),
             Wo, bo.reshape(1, 1))
  return out
